# baseline (speedup 1.0000x reference)
import functools

import jax
import jax.numpy as jnp
from jax.experimental import pallas as pl

N = 1_000_000
ELE_DIM = 32
MAT_FEAT = 83
TILE = 8_000


def _decoder_body(x_ref, wt_ref, b_ref, out_ref, mask_ref):
    x = x_ref[...]
    mask = jnp.any(x != 0.0, axis=1)
    y = jnp.dot(x, wt_ref[...], preferred_element_type=jnp.float32)
    y = y + b_ref[...]
    y = y * 0.5 * (1.0 + jax.lax.erf(y * 0.7071067811865476))
    y = jnp.where(mask[:, None], y, 0.0)
    out_ref[...] = jax.nn.sigmoid(y)
    mask_ref[...] = mask[:, None]


@functools.partial(jax.jit, static_argnames=("interpret",))
def _decoder(inputs, wt, b2, interpret=False):
    n = inputs.shape[0]
    grid = (n // TILE,)
    out, mask = pl.pallas_call(
        _decoder_body,
        grid=grid,
        in_specs=[
            pl.BlockSpec((TILE, ELE_DIM), lambda i: (i, 0)),
            pl.BlockSpec((ELE_DIM, MAT_FEAT), lambda i: (0, 0)),
            pl.BlockSpec((1, MAT_FEAT), lambda i: (0, 0)),
        ],
        out_specs=[
            pl.BlockSpec((TILE, MAT_FEAT), lambda i: (i, 0)),
            pl.BlockSpec((TILE, 1), lambda i: (i, 0)),
        ],
        out_shape=[
            jax.ShapeDtypeStruct((n, MAT_FEAT), jnp.float32),
            jax.ShapeDtypeStruct((n, 1), jnp.bool_),
        ],
        interpret=interpret,
    )(inputs, wt, b2)
    return out, mask.reshape(n)


def kernel(inputs, W, b):
    wt = W.T
    b2 = b.reshape(1, MAT_FEAT)
    return _decoder(inputs, wt, b2)
